# Initial kernel scaffold; baseline (speedup 1.0000x reference)
import jax
import jax.numpy as jnp
from jax.experimental import pallas as pl


def kernel(x, fw0, fw1, fw2, hw0, hw1, hw2, lw0, lw1, lw2):
    raise NotImplementedError("write your pallas kernel here")


# speedup vs baseline: 2.3200x; 2.3200x over previous
import jax
import jax.numpy as jnp
from jax.experimental import pallas as pl

WI, WH, WO = 5, 4, 1
INPUT_DIM, DEPTH, WIDTH, OUT = 128, 6, 16, 16
BATCH = 32
NF = INPUT_DIM * WIDTH
NH = WIDTH * WIDTH
NL = WIDTH * OUT
NHID = DEPTH - 2
SEG_F = NF // WIDTH


_SPLIT = 65537.0


def _rnd(v):
    c = v * _SPLIT
    return c - (c - v)


def _tree_sum(v, axis):
    n = v.shape[axis]
    while n > 1:
        h = n // 2
        v = jax.lax.slice_in_dim(v, 0, h, axis=axis) + \
            jax.lax.slice_in_dim(v, h, 2 * h, axis=axis)
        n = h
    return jnp.squeeze(v, axis=axis)


def _first_layer_body(x_ref, a_ref, w1_ref, w2_ref, out_ref):
    g = pl.program_id(0)
    tb = _rnd(x_ref[:])
    pcs = []
    for c in range(WH):
        ac = a_ref[c, :]
        pcs.append(_rnd(ac[None, :, None] * tb[:, None, :]))
    h3 = None
    for r in range(WH):
        h2 = None
        for c in range(WH):
            term = w1_ref[r, c, :][None, :, None] * pcs[c]
            h2 = term if h2 is None else h2 + term
        h3r = w2_ref[r, :][None, :, None] * _rnd(h2)
        h3 = h3r if h3 is None else h3 + h3r
    out_ref[pl.ds(g, 1), :] = _tree_sum(_tree_sum(h3, 2), 1)[None, :]


def _layer16(t, a, w1, w2):
    tb = _rnd(t)
    pcs = [_rnd(a[c][None, None, :] * tb[:, :, None]) for c in range(WH)]
    h3 = None
    for r in range(WH):
        h2 = None
        for c in range(WH):
            term = w1[r, c][None, None, :] * pcs[c]
            h2 = term if h2 is None else h2 + term
        h3r = w2[r][None, None, :] * _rnd(h2)
        h3 = h3r if h3 is None else h3 + h3r
    hsum = _tree_sum(h3, 1)
    return _tree_sum(hsum.reshape(BATCH, WIDTH, WIDTH), 2)


def _tail_body(t_ref, ah_ref, w1h_ref, w2h_ref, al_ref, w1l_ref, w2l_ref,
               out_ref):
    t = t_ref[:]
    residual = t
    for i in range(NHID):
        if i % 2 == 0:
            residual = t
        t = _layer16(t, ah_ref[i], w1h_ref[i], w2h_ref[i])
        if i % 2 != 0:
            t = t + residual
    out_ref[:] = _layer16(t, al_ref[:], w1l_ref[:], w2l_ref[:])


def _pr(v):
    return _rnd(v)


def kernel(x, fw0, fw1, fw2, hw0, hw1, hw2, lw0, lw1, lw2):
    af = _pr(fw0.reshape(NF, WH, WI)[:, :, WI - 1]).T
    w1f = _pr(fw1.reshape(NF, WH, WH)).transpose(1, 2, 0)
    w2f = _pr(fw2.reshape(NF, WO * WH)).T
    ah = _pr(hw0.reshape(NHID, NH, WH, WI)[..., WI - 1]).transpose(0, 2, 1)
    w1h = _pr(hw1.reshape(NHID, NH, WH, WH)).transpose(0, 2, 3, 1)
    w2h = _pr(hw2.reshape(NHID, NH, WH)).transpose(0, 2, 1)
    al = _pr(lw0.reshape(NL, WH, WI)[:, :, WI - 1]).T
    w1l = _pr(lw1.reshape(NL, WH, WH)).transpose(1, 2, 0)
    w2l = _pr(lw2.reshape(NL, WH)).T

    t1_t = pl.pallas_call(
        _first_layer_body,
        grid=(WIDTH,),
        in_specs=[
            pl.BlockSpec((BATCH, INPUT_DIM), lambda g: (0, 0)),
            pl.BlockSpec((WH, SEG_F), lambda g: (0, g)),
            pl.BlockSpec((WH, WH, SEG_F), lambda g: (0, 0, g)),
            pl.BlockSpec((WH, SEG_F), lambda g: (0, g)),
        ],
        out_specs=pl.BlockSpec((WIDTH, BATCH), lambda g: (0, 0)),
        out_shape=jax.ShapeDtypeStruct((WIDTH, BATCH), jnp.float32),
    )(x, af, w1f, w2f)

    return pl.pallas_call(
        _tail_body,
        out_shape=jax.ShapeDtypeStruct((BATCH, OUT), jnp.float32),
    )(t1_t.T, ah, w1h, w2h, al, w1l, w2l)


# speedup vs baseline: 2.5477x; 1.0981x over previous
import jax
import jax.numpy as jnp
from jax.experimental import pallas as pl
from jax.experimental.pallas import tpu as pltpu

WI, WH, WO = 5, 4, 1
INPUT_DIM, DEPTH, WIDTH, OUT = 128, 6, 16, 16
BATCH = 32
NF = INPUT_DIM * WIDTH
NH = WIDTH * WIDTH
NL = WIDTH * OUT
NHID = DEPTH - 2
SEG_F = NF // WIDTH


def _rnd(v):
    c = v * 65537.0
    return c - (c - v)


def _tree_sum(v, axis):
    n = v.shape[axis]
    while n > 1:
        h = n // 2
        v = jax.lax.slice_in_dim(v, 0, h, axis=axis) + \
            jax.lax.slice_in_dim(v, h, 2 * h, axis=axis)
        n = h
    return jnp.squeeze(v, axis=axis)


def _layer16(t, a, w1, w2):
    tb = _rnd(t)
    pcs = [_rnd(a[c][None, None, :] * tb[:, :, None]) for c in range(WH)]
    h3 = None
    for r in range(WH):
        h2 = None
        for c in range(WH):
            term = w1[r, c][None, None, :] * pcs[c]
            h2 = term if h2 is None else h2 + term
        h3r = w2[r][None, None, :] * _rnd(h2)
        h3 = h3r if h3 is None else h3 + h3r
    hsum = _tree_sum(h3, 1)
    return _tree_sum(hsum.reshape(BATCH, WIDTH, WIDTH), 2)


def _body(x_ref, w0_ref, w1_ref, w2_ref, ah_ref, w1h_ref, w2h_ref,
          al_ref, w1l_ref, w2l_ref, out_ref, acc_ref):
    g = pl.program_id(0)

    @pl.when(g < WIDTH)
    def _segment():
        tb = _rnd(x_ref[:])
        pcs = []
        for c in range(WH):
            ac = _rnd(w0_ref[:, WI * c + WI - 1])
            pcs.append(_rnd(ac[None, :, None] * tb[:, None, :]))
        h3 = None
        for r in range(WH):
            h2 = None
            for c in range(WH):
                term = _rnd(w1_ref[:, WH * r + c])[None, :, None] * pcs[c]
                h2 = term if h2 is None else h2 + term
            h3r = _rnd(w2_ref[:, r])[None, :, None] * _rnd(h2)
            h3 = h3r if h3 is None else h3 + h3r
        acc_ref[pl.ds(g, 1), :] = _tree_sum(_tree_sum(h3, 2), 1)[None, :]

    @pl.when(g == WIDTH)
    def _tail():
        t = acc_ref[:].T
        residual = t
        for i in range(NHID):
            if i % 2 == 0:
                residual = t
            t = _layer16(t, _rnd(ah_ref[i]), _rnd(w1h_ref[i]), _rnd(w2h_ref[i]))
            if i % 2 != 0:
                t = t + residual
        out_ref[:] = _layer16(t, _rnd(al_ref[:]), _rnd(w1l_ref[:]), _rnd(w2l_ref[:]))


def kernel(x, fw0, fw1, fw2, hw0, hw1, hw2, lw0, lw1, lw2):
    w0 = fw0.reshape(NF, WH * WI)
    w1 = fw1.reshape(NF, WH * WH)
    w2 = fw2.reshape(NF, WO * WH)
    ah = hw0.reshape(NHID, NH, WH, WI)[..., WI - 1].transpose(0, 2, 1)
    w1h = hw1.reshape(NHID, NH, WH, WH).transpose(0, 2, 3, 1)
    w2h = hw2.reshape(NHID, NH, WH).transpose(0, 2, 1)
    al = lw0.reshape(NL, WH, WI)[:, :, WI - 1].T
    w1l = lw1.reshape(NL, WH, WH).transpose(1, 2, 0)
    w2l = lw2.reshape(NL, WH).T

    last = lambda g: jnp.minimum(g, WIDTH - 1)
    return pl.pallas_call(
        _body,
        grid=(WIDTH + 1,),
        in_specs=[
            pl.BlockSpec((BATCH, INPUT_DIM), lambda g: (0, 0)),
            pl.BlockSpec((SEG_F, WH * WI), lambda g: (last(g), 0)),
            pl.BlockSpec((SEG_F, WH * WH), lambda g: (last(g), 0)),
            pl.BlockSpec((SEG_F, WO * WH), lambda g: (last(g), 0)),
            pl.BlockSpec((NHID, WH, NH), lambda g: (0, 0, 0)),
            pl.BlockSpec((NHID, WH, WH, NH), lambda g: (0, 0, 0, 0)),
            pl.BlockSpec((NHID, WH, NH), lambda g: (0, 0, 0)),
            pl.BlockSpec((WH, NL), lambda g: (0, 0)),
            pl.BlockSpec((WH, WH, NL), lambda g: (0, 0, 0)),
            pl.BlockSpec((WH, NL), lambda g: (0, 0)),
        ],
        out_specs=pl.BlockSpec((BATCH, OUT), lambda g: (0, 0)),
        out_shape=jax.ShapeDtypeStruct((BATCH, OUT), jnp.float32),
        scratch_shapes=[pltpu.VMEM((WIDTH, BATCH), jnp.float32)],
    )(x, w0, w1, w2, ah, w1h, w2h, al, w1l, w2l)


# speedup vs baseline: 9.5148x; 3.7346x over previous
import jax
import jax.numpy as jnp
from jax.experimental import pallas as pl

WI, WH, WO = 5, 4, 1
INPUT_DIM, DEPTH, WIDTH, OUT = 128, 6, 16, 16
BATCH = 32
NF = INPUT_DIM * WIDTH
NH = WIDTH * WIDTH
NL = WIDTH * OUT
NHID = DEPTH - 2
NM = 128


def _rnd(v):
    c = v * 65537.0
    return c - (c - v)


def _tree_sum(v, axis):
    n = v.shape[axis]
    while n > 1:
        h = n // 2
        v = jax.lax.slice_in_dim(v, 0, h, axis=axis) + \
            jax.lax.slice_in_dim(v, h, 2 * h, axis=axis)
        n = h
    return jnp.squeeze(v, axis=axis)


def _layer16(t, a, w1, w2):
    tb = _rnd(t)
    pcs = [_rnd(a[c][None, None, :] * tb[:, :, None]) for c in range(WH)]
    h3 = None
    for r in range(WH):
        h2 = None
        for c in range(WH):
            term = w1[r, c][None, None, :] * pcs[c]
            h2 = term if h2 is None else h2 + term
        h3r = w2[r][None, None, :] * _rnd(h2)
        h3 = h3r if h3 is None else h3 + h3r
    hsum = _tree_sum(h3, 1)
    return _tree_sum(hsum.reshape(BATCH, WIDTH, WIDTH), 2)


def _body(x_ref, w0_ref, w1_ref, w2_ref, ah_ref, w1h_ref, w2h_ref,
          al_ref, w1l_ref, w2l_ref, out_ref):
    xhat = 1.0 + jax.lax.broadcasted_iota(
        jnp.int32, (1, NM), 1).astype(jnp.float32) * (1.0 / NM)

    phat = []
    for c in range(WH):
        a_c = _rnd(w0_ref[:, WI * c + WI - 1:WI * c + WI])
        phat.append(_rnd(a_c * xhat))
    h = None
    for r in range(WH):
        g = None
        for c in range(WH):
            term = _rnd(w1_ref[:, WH * r + c:WH * r + c + 1]) * phat[c]
            g = term if g is None else g + term
        hr = _rnd(w2_ref[:, r:r + 1]) * _rnd(g)
        h = hr if h is None else h + hr
    seg_a = _tree_sum(h.reshape(WIDTH, NF // WIDTH, NM), 1)

    u = jax.lax.bitcast_convert_type(x_ref[:], jnp.uint32)
    r_ = u + jnp.uint32(0x7FFF) + ((u >> 16) & jnp.uint32(1))
    mi = ((r_ >> 16) & jnp.uint32(0x7F)).astype(jnp.int32)
    scale = jax.lax.bitcast_convert_type(r_ & jnp.uint32(0xFF800000), jnp.float32)
    miota = jax.lax.broadcasted_iota(jnp.int32, (1, 1, NM), 2)
    onehot = jnp.where(mi[:, :, None] == miota, scale[:, :, None], 0.0)
    s = _tree_sum(onehot, 1)
    t1 = _tree_sum(s[:, None, :] * seg_a[None, :, :], 2)

    t = t1
    residual = t
    for i in range(NHID):
        if i % 2 == 0:
            residual = t
        t = _layer16(t, _rnd(ah_ref[i]), _rnd(w1h_ref[i]), _rnd(w2h_ref[i]))
        if i % 2 != 0:
            t = t + residual
    out_ref[:] = _layer16(t, _rnd(al_ref[:]), _rnd(w1l_ref[:]), _rnd(w2l_ref[:]))


def kernel(x, fw0, fw1, fw2, hw0, hw1, hw2, lw0, lw1, lw2):
    w0 = fw0.reshape(NF, WH * WI)
    w1 = fw1.reshape(NF, WH * WH)
    w2 = fw2.reshape(NF, WO * WH)
    ah = hw0.reshape(NHID, NH, WH, WI)[..., WI - 1].transpose(0, 2, 1)
    w1h = hw1.reshape(NHID, NH, WH, WH).transpose(0, 2, 3, 1)
    w2h = hw2.reshape(NHID, NH, WH).transpose(0, 2, 1)
    al = lw0.reshape(NL, WH, WI)[:, :, WI - 1].T
    w1l = lw1.reshape(NL, WH, WH).transpose(1, 2, 0)
    w2l = lw2.reshape(NL, WH).T

    return pl.pallas_call(
        _body,
        out_shape=jax.ShapeDtypeStruct((BATCH, OUT), jnp.float32),
    )(x, w0, w1, w2, ah, w1h, w2h, al, w1l, w2l)
